# baseline (speedup 1.0000x reference)
import functools

import jax
import jax.numpy as jnp
from jax import lax
from jax.experimental import pallas as pl
from jax.experimental.pallas import tpu as pltpu
from jax.experimental.pallas import tpu_sc as plsc

N = 10000
E = 320000
D = 128
H = 256
G = 64
C = 150

NC = 2
NS = 16
NW = NC * NS
CH = 128
RING = 2
CPT = 80
HALF = CPT // 2
PHASE_PAIRS = HALF // RING
EPT = CPT * CH
E_PAD = NW * EPT
SPT = 632
NP = SPT * NS


def _sc_scatter_add_body(x_hbm, edges_hbm, zeros_hbm, out_hbm, src_v, dst_v,
                         rows0, rows1, agg_sh, gsem0, gsem1, ssem0, ssem1):
    rows_l = [rows0, rows1]
    gsem_l = [gsem0, gsem1]
    ssem_l = [ssem0, ssem1]
    c = lax.axis_index("c")
    s = lax.axis_index("s")
    wid = c * NS + s

    base_row = s * SPT
    pltpu.sync_copy(zeros_hbm, agg_sh.at[pl.ds(base_row, SPT)])

    pltpu.sync_copy(edges_hbm.at[0].at[wid].at[pl.ds(0, HALF)], src_v)
    pltpu.sync_copy(edges_hbm.at[1].at[wid].at[pl.ds(0, HALF)], dst_v)
    plsc.subcore_barrier()

    def _wait(src, dst, sem):
        pltpu.make_async_copy(src, dst, sem).wait()

    def _ring_body(jj, _):
        for b in range(RING):
            @pl.when(jj > 0)
            def _():
                _wait(rows_l[b], agg_sh.at[dst_v.at[0]], ssem_l[b])

        @pl.when(jj == PHASE_PAIRS)
        def _():
            pltpu.sync_copy(edges_hbm.at[0].at[wid].at[pl.ds(HALF, HALF)],
                            src_v)
            pltpu.sync_copy(edges_hbm.at[1].at[wid].at[pl.ds(HALF, HALF)],
                            dst_v)

        base = lax.rem(jj, PHASE_PAIRS) * RING
        for b in range(RING):
            pltpu.async_copy(x_hbm.at[src_v.at[base + b]], rows_l[b],
                             gsem_l[b])
        for b in range(RING):
            _wait(x_hbm.at[src_v.at[0]], rows_l[b], gsem_l[b])
            pltpu.async_copy(rows_l[b], agg_sh.at[dst_v.at[base + b]],
                             ssem_l[b], add=True)
        return 0

    lax.fori_loop(0, CPT // RING, _ring_body, 0)
    for b in range(RING):
        _wait(rows_l[b], agg_sh.at[dst_v.at[0]], ssem_l[b])
    plsc.subcore_barrier()

    pltpu.sync_copy(agg_sh.at[pl.ds(base_row, SPT)],
                    out_hbm.at[c].at[pl.ds(base_row, SPT)])


@functools.cache
def _sc_scatter_add():
    mesh = plsc.VectorSubcoreMesh(core_axis_name="c", subcore_axis_name="s",
                                  num_cores=NC, num_subcores=NS)
    return pl.kernel(
        _sc_scatter_add_body,
        out_type=jax.ShapeDtypeStruct((NC, NP, D), jnp.float32),
        mesh=mesh,
        scratch_types=[
            pltpu.VMEM((HALF, CH), jnp.int32),
            pltpu.VMEM((HALF, CH), jnp.int32),
            pltpu.VMEM((CH, D), jnp.float32),
            pltpu.VMEM((CH, D), jnp.float32),
            pltpu.VMEM_SHARED((NP, D), jnp.float32),
            pltpu.SemaphoreType.DMA,
            pltpu.SemaphoreType.DMA,
            pltpu.SemaphoreType.DMA,
            pltpu.SemaphoreType.DMA,
        ],
    )


BN = 1000
NB = N // BN


def _tc_fused_body(x_b, a0_b, a1_b, wg, bg, wc, bc, batch_b, wt_bf, bt, wo_bf,
                   bo, out_b, h_s, gmax_s, z_s):
    p = pl.program_id(0)
    i = pl.program_id(1)

    @pl.when(p == 0)
    def _phase0():
        xs = (x_b[...] + a0_b[0] + a1_b[0]).astype(jnp.bfloat16)
        h = jnp.dot(xs, wg[...],
                    preferred_element_type=jnp.float32) + bg[...]
        h_s[pl.ds(i * BN, BN), :] = h
        xg = jnp.maximum(
            jnp.dot(h.astype(jnp.bfloat16), wc[...],
                    preferred_element_type=jnp.float32) + bc[...],
            0.0)

        @pl.when(i == 0)
        def _():
            gmax_s[...] = jnp.zeros_like(gmax_s)

        bcol = batch_b[...]
        gmin = jnp.min(bcol).astype(jnp.int32)
        gmax_id = jnp.max(bcol).astype(jnp.int32)

        def _g_body(g, _):
            mask = bcol == g.astype(jnp.float32)
            m = jnp.max(jnp.where(mask, xg, 0.0), axis=0, keepdims=True)
            gmax_s[pl.ds(g, 1), :] = jnp.maximum(gmax_s[pl.ds(g, 1), :], m)
            return 0

        lax.fori_loop(gmin, gmax_id + 1, _g_body, 0)

    @pl.when(p == 1)
    def _phase1():
        @pl.when(i == 0)
        def _():
            z_s[...] = jnp.dot(gmax_s[...].astype(jnp.bfloat16),
                               wt_bf[pl.ds(H, H), :],
                               preferred_element_type=jnp.float32) + bt[...]

        bcol = batch_b[...]
        onehot = (bcol == lax.broadcasted_iota(jnp.int32, (1, G), 1)
                  .astype(jnp.float32)).astype(jnp.float32)
        zb = jnp.dot(onehot, z_s[...], preferred_element_type=jnp.float32)
        h_b = h_s[pl.ds(i * BN, BN), :]
        t = jnp.maximum(
            jnp.dot(h_b.astype(jnp.bfloat16), wt_bf[pl.ds(0, H), :],
                    preferred_element_type=jnp.float32) + zb, 0.0)
        out_b[...] = jnp.dot(t.astype(jnp.bfloat16), wo_bf[...],
                             preferred_element_type=jnp.float32) + bo[...]


def kernel(x, edge_indices, batch, W_gcn, b_gcn, W_conv, b_conv, W_t, b_t,
           W_out, b_out):
    pad = E_PAD - E
    pad_idx = jnp.arange(pad, dtype=jnp.int32)
    pads2 = jnp.stack([(pad_idx * 131) % N, N + pad_idx % (NP - N)])
    edges4 = jnp.concatenate([edge_indices, pads2],
                             axis=1).reshape(2, NW, CPT, CH)

    zeros_slab = jnp.zeros((SPT, D), jnp.float32)
    agg2 = _sc_scatter_add()(x, edges4, zeros_slab)

    batch_col = batch.astype(jnp.float32).reshape(N, 1)

    out = pl.pallas_call(
        _tc_fused_body,
        grid=(2, NB),
        in_specs=[
            pl.BlockSpec((BN, D), lambda p, i: (jnp.where(p == 0, i, 0), 0)),
            pl.BlockSpec((1, BN, D),
                         lambda p, i: (0, jnp.where(p == 0, i, 0), 0)),
            pl.BlockSpec((1, BN, D),
                         lambda p, i: (1, jnp.where(p == 0, i, 0), 0)),
            pl.BlockSpec((D, H), lambda p, i: (0, 0)),
            pl.BlockSpec((1, H), lambda p, i: (0, 0)),
            pl.BlockSpec((H, H), lambda p, i: (0, 0)),
            pl.BlockSpec((1, H), lambda p, i: (0, 0)),
            pl.BlockSpec((BN, 1), lambda p, i: (i, 0)),
            pl.BlockSpec((2 * H, 1024), lambda p, i: (0, 0)),
            pl.BlockSpec((1, 1024), lambda p, i: (0, 0)),
            pl.BlockSpec((1024, C), lambda p, i: (0, 0)),
            pl.BlockSpec((1, C), lambda p, i: (0, 0)),
        ],
        out_specs=pl.BlockSpec((BN, C),
                               lambda p, i: (jnp.where(p == 1, i, 0), 0)),
        out_shape=jax.ShapeDtypeStruct((N, C), jnp.float32),
        scratch_shapes=[
            pltpu.VMEM((N, H), jnp.float32),
            pltpu.VMEM((G, H), jnp.float32),
            pltpu.VMEM((G, 1024), jnp.float32),
        ],
    )(x, agg2, agg2, W_gcn.astype(jnp.bfloat16), b_gcn.reshape(1, H),
      W_conv.astype(jnp.bfloat16), b_conv.reshape(1, H), batch_col,
      W_t.astype(jnp.bfloat16), b_t.reshape(1, 1024),
      W_out.astype(jnp.bfloat16), b_out.reshape(1, C))

    return out


# speedup vs baseline: 1.0143x; 1.0143x over previous
import functools

import jax
import jax.numpy as jnp
from jax import lax
from jax.experimental import pallas as pl
from jax.experimental.pallas import tpu as pltpu
from jax.experimental.pallas import tpu_sc as plsc

N = 10000
E = 320000
D = 128
H = 256
G = 64
C = 150

NC = 2
NS = 16
NW = NC * NS
CH = 128
RING = 2
CPT = 80
HALF = CPT // 2
PHASE_PAIRS = HALF // RING
EPT = CPT * CH
E_PAD = NW * EPT
SPT = 632
NP = SPT * NS


def _sc_scatter_add_body(x_hbm, edges_hbm, out_hbm, src_v, dst_v,
                         rows0, rows1, agg_sh, gsem0, gsem1, ssem0, ssem1):
    rows_l = [rows0, rows1]
    gsem_l = [gsem0, gsem1]
    ssem_l = [ssem0, ssem1]
    c = lax.axis_index("c")
    s = lax.axis_index("s")
    wid = c * NS + s

    def _zero_body(i, _):
        r = i // (D // 16)
        col = lax.rem(i, D // 16)
        rows0[r, pl.ds(col * 16, 16)] = jnp.zeros((16,), jnp.float32)
        return 0

    lax.fori_loop(0, CH * (D // 16), _zero_body, 0)

    base_row = s * SPT
    n_full = SPT // CH
    rem = SPT - n_full * CH
    for k in range(n_full):
        pltpu.sync_copy(rows0, agg_sh.at[pl.ds(base_row + k * CH, CH)])
    pltpu.sync_copy(rows0.at[pl.ds(0, rem)],
                    agg_sh.at[pl.ds(base_row + n_full * CH, rem)])

    pltpu.sync_copy(edges_hbm.at[0].at[wid].at[pl.ds(0, HALF)], src_v)
    pltpu.sync_copy(edges_hbm.at[1].at[wid].at[pl.ds(0, HALF)], dst_v)
    plsc.subcore_barrier()

    def _wait(src, dst, sem):
        pltpu.make_async_copy(src, dst, sem).wait()

    def _ring_body(jj, _):
        for b in range(RING):
            @pl.when(jj > 0)
            def _():
                _wait(rows_l[b], agg_sh.at[dst_v.at[0]], ssem_l[b])

        @pl.when(jj == PHASE_PAIRS)
        def _():
            pltpu.sync_copy(edges_hbm.at[0].at[wid].at[pl.ds(HALF, HALF)],
                            src_v)
            pltpu.sync_copy(edges_hbm.at[1].at[wid].at[pl.ds(HALF, HALF)],
                            dst_v)

        base = lax.rem(jj, PHASE_PAIRS) * RING
        for b in range(RING):
            pltpu.async_copy(x_hbm.at[src_v.at[base + b]], rows_l[b],
                             gsem_l[b])
        for b in range(RING):
            _wait(x_hbm.at[src_v.at[0]], rows_l[b], gsem_l[b])
            pltpu.async_copy(rows_l[b], agg_sh.at[dst_v.at[base + b]],
                             ssem_l[b], add=True)
        return 0

    lax.fori_loop(0, CPT // RING, _ring_body, 0)
    for b in range(RING):
        _wait(rows_l[b], agg_sh.at[dst_v.at[0]], ssem_l[b])
    plsc.subcore_barrier()

    pltpu.sync_copy(agg_sh.at[pl.ds(base_row, SPT)],
                    out_hbm.at[c].at[pl.ds(base_row, SPT)])


@functools.cache
def _sc_scatter_add():
    mesh = plsc.VectorSubcoreMesh(core_axis_name="c", subcore_axis_name="s",
                                  num_cores=NC, num_subcores=NS)
    return pl.kernel(
        _sc_scatter_add_body,
        out_type=jax.ShapeDtypeStruct((NC, NP, D), jnp.float32),
        mesh=mesh,
        scratch_types=[
            pltpu.VMEM((HALF, CH), jnp.int32),
            pltpu.VMEM((HALF, CH), jnp.int32),
            pltpu.VMEM((CH, D), jnp.float32),
            pltpu.VMEM((CH, D), jnp.float32),
            pltpu.VMEM_SHARED((NP, D), jnp.float32),
            pltpu.SemaphoreType.DMA,
            pltpu.SemaphoreType.DMA,
            pltpu.SemaphoreType.DMA,
            pltpu.SemaphoreType.DMA,
        ],
    )


BN = 1000
NB = N // BN


def _tc_fused_body(x_b, a0_b, a1_b, wg, bg, wc, bc, batch_b, wt_bf, bt, wo_bf,
                   bo, out_b, h_s, gmax_s, z_s):
    p = pl.program_id(0)
    i = pl.program_id(1)

    @pl.when(p == 0)
    def _phase0():
        xs = x_b[...] + a0_b[0] + a1_b[0]
        h = jnp.dot(xs, wg[...], preferred_element_type=jnp.float32) + bg[...]
        h_s[pl.ds(i * BN, BN), :] = h
        xg = jnp.maximum(
            jnp.dot(h, wc[...], preferred_element_type=jnp.float32) + bc[...],
            0.0)

        @pl.when(i == 0)
        def _():
            gmax_s[...] = jnp.zeros_like(gmax_s)

        bcol = batch_b[...]
        gmin = jnp.min(bcol).astype(jnp.int32)
        gmax_id = jnp.max(bcol).astype(jnp.int32)

        def _g_body(g, _):
            mask = bcol == g.astype(jnp.float32)
            m = jnp.max(jnp.where(mask, xg, 0.0), axis=0, keepdims=True)
            gmax_s[pl.ds(g, 1), :] = jnp.maximum(gmax_s[pl.ds(g, 1), :], m)
            return 0

        lax.fori_loop(gmin, gmax_id + 1, _g_body, 0)

    @pl.when(p == 1)
    def _phase1():
        @pl.when(i == 0)
        def _():
            z_s[...] = jnp.dot(gmax_s[...].astype(jnp.bfloat16),
                               wt_bf[pl.ds(H, H), :],
                               preferred_element_type=jnp.float32) + bt[...]

        bcol = batch_b[...]
        onehot = (bcol == lax.broadcasted_iota(jnp.int32, (1, G), 1)
                  .astype(jnp.float32)).astype(jnp.float32)
        zb = jnp.dot(onehot, z_s[...], preferred_element_type=jnp.float32)
        h_b = h_s[pl.ds(i * BN, BN), :]
        t = jnp.maximum(
            jnp.dot(h_b.astype(jnp.bfloat16), wt_bf[pl.ds(0, H), :],
                    preferred_element_type=jnp.float32) + zb, 0.0)
        out_b[...] = jnp.dot(t.astype(jnp.bfloat16), wo_bf[...],
                             preferred_element_type=jnp.float32) + bo[...]


def kernel(x, edge_indices, batch, W_gcn, b_gcn, W_conv, b_conv, W_t, b_t,
           W_out, b_out):
    pad = E_PAD - E
    pad_idx = jnp.arange(pad, dtype=jnp.int32)
    pads2 = jnp.stack([(pad_idx * 131) % N, N + pad_idx % (NP - N)])
    edges4 = jnp.concatenate([edge_indices, pads2],
                             axis=1).reshape(2, NW, CPT, CH)

    agg2 = _sc_scatter_add()(x, edges4)

    batch_col = batch.astype(jnp.float32).reshape(N, 1)

    out = pl.pallas_call(
        _tc_fused_body,
        grid=(2, NB),
        in_specs=[
            pl.BlockSpec((BN, D), lambda p, i: (jnp.where(p == 0, i, 0), 0)),
            pl.BlockSpec((1, BN, D),
                         lambda p, i: (0, jnp.where(p == 0, i, 0), 0)),
            pl.BlockSpec((1, BN, D),
                         lambda p, i: (1, jnp.where(p == 0, i, 0), 0)),
            pl.BlockSpec((D, H), lambda p, i: (0, 0)),
            pl.BlockSpec((1, H), lambda p, i: (0, 0)),
            pl.BlockSpec((H, H), lambda p, i: (0, 0)),
            pl.BlockSpec((1, H), lambda p, i: (0, 0)),
            pl.BlockSpec((BN, 1), lambda p, i: (i, 0)),
            pl.BlockSpec((2 * H, 1024), lambda p, i: (0, 0)),
            pl.BlockSpec((1, 1024), lambda p, i: (0, 0)),
            pl.BlockSpec((1024, C), lambda p, i: (0, 0)),
            pl.BlockSpec((1, C), lambda p, i: (0, 0)),
        ],
        out_specs=pl.BlockSpec((BN, C),
                               lambda p, i: (jnp.where(p == 1, i, 0), 0)),
        out_shape=jax.ShapeDtypeStruct((N, C), jnp.float32),
        scratch_shapes=[
            pltpu.VMEM((N, H), jnp.float32),
            pltpu.VMEM((G, H), jnp.float32),
            pltpu.VMEM((G, 1024), jnp.float32),
        ],
    )(x, agg2, agg2, W_gcn, b_gcn.reshape(1, H), W_conv,
      b_conv.reshape(1, H), batch_col, W_t.astype(jnp.bfloat16),
      b_t.reshape(1, 1024), W_out.astype(jnp.bfloat16), b_out.reshape(1, C))

    return out
